# baseline (speedup 1.0000x reference)
import functools

import jax
import jax.numpy as jnp
from jax import lax
from jax.experimental import pallas as pl
from jax.experimental.pallas import tpu as pltpu
from jax.experimental.pallas import tpu_sc as plsc

D_MODEL = 64
_PAD = 128
_NC = 2
_NS = 16
_NW = _NC * _NS
_SUB = 128
_K = 4
_CHUNK = _K * _SUB
_IDXROWS = 8
_L = 16


@functools.partial(jax.jit, static_argnames=("n_iter",))
def _gather(table_pad, idx2d, n_iter):
    b = idx2d.shape[0] * idx2d.shape[1]
    b_per_w = b // _NW
    mesh = plsc.VectorSubcoreMesh(core_axis_name="c", subcore_axis_name="s")

    @functools.partial(
        pl.kernel,
        mesh=mesh,
        out_type=jax.ShapeDtypeStruct((b // 2, _PAD), jnp.float32),
        scratch_types=[
            pltpu.VMEM((_IDXROWS, _SUB), jnp.int32),
            pltpu.VMEM((_CHUNK, _PAD), jnp.float32),
            pltpu.VMEM((_CHUNK // 2, _PAD), jnp.float32),
            pltpu.SemaphoreType.DMA,
        ],
    )
    def gather_kernel(table_hbm, idx_hbm, out_hbm, idx_v, rows_v, comp_v, sem):
        wid = lax.axis_index("s") * _NC + lax.axis_index("c")
        base = wid * b_per_w

        def compact(r0, carry):
            for dr in range(8):
                for c in range(D_MODEL // _L):
                    comp_v[r0 * 4 + dr // 2, pl.ds((dr % 2) * D_MODEL + c * _L, _L)] = (
                        rows_v[r0 * 8 + dr, pl.ds(c * _L, _L)]
                    )
            return carry

        def half(off, j0):
            copies = [
                pltpu.async_copy(
                    table_hbm.at[idx_v.at[j0 + j]],
                    rows_v.at[pl.ds(j * _SUB, _SUB)],
                    sem,
                )
                for j in range(_K)
            ]
            for c in copies:
                c.wait()
            lax.fori_loop(0, _CHUNK // 8, compact, 0)
            off2 = pl.multiple_of(off // 2, _CHUNK // 2)
            pltpu.sync_copy(comp_v, out_hbm.at[pl.ds(off2, _CHUNK // 2)])

        def body(i, carry):
            off = pl.multiple_of(base + i * (2 * _CHUNK), 2 * _CHUNK)
            row0 = pl.multiple_of(off // _SUB, _IDXROWS)
            pltpu.sync_copy(idx_hbm.at[pl.ds(row0, _IDXROWS)], idx_v)
            half(off, 0)
            half(off + _CHUNK, _K)
            return carry

        lax.fori_loop(0, n_iter, body, 0)

    return gather_kernel(table_pad, idx2d)


def kernel(x, table):
    s0, s1 = x.shape
    b = s0 * s1
    idx2d = x.reshape(b // _SUB, _SUB).astype(jnp.int32)
    table_pad = jnp.pad(table, ((0, 0), (0, _PAD - D_MODEL)))
    n_iter = b // (_NW * 2 * _CHUNK)
    out = _gather(table_pad, idx2d, n_iter)
    return out.reshape(s0, s1, D_MODEL)


# speedup vs baseline: 1.5283x; 1.5283x over previous
import functools

import jax
import jax.numpy as jnp
from jax import lax
from jax.experimental import pallas as pl
from jax.experimental.pallas import tpu as pltpu
from jax.experimental.pallas import tpu_sc as plsc

D_MODEL = 64
_NC = 2
_NS = 16
_NW = _NC * _NS
_MAJ = 4
_IDXSTAGE = 3200


@functools.partial(jax.jit, static_argnames=("s0", "s1"))
def _gather(table, idx1d, s0, s1):
    b = s0 * s1
    maj_per_w = s0 // _NW
    n_outer = maj_per_w * s1 // _IDXSTAGE
    mesh = plsc.VectorSubcoreMesh(core_axis_name="c", subcore_axis_name="s")

    @functools.partial(
        pl.kernel,
        mesh=mesh,
        compiler_params=pltpu.CompilerParams(use_tc_tiling_on_sc=False),
        out_type=jax.ShapeDtypeStruct((s0, s1, D_MODEL), jnp.float32),
        scratch_types=[
            pltpu.VMEM((_IDXSTAGE,), jnp.int32),
            pltpu.VMEM((_MAJ, 200, D_MODEL), jnp.float32),
            pltpu.SemaphoreType.DMA,
        ],
    )
    def gather_kernel(table_hbm, idx_hbm, out_hbm, idx_v, rows_v, sem):
        wid = lax.axis_index("s") * _NC + lax.axis_index("c")
        maj_base = wid * maj_per_w

        def body(t, carry):
            flat0 = pl.multiple_of((maj_base + t * 16) * s1, _IDXSTAGE)
            pltpu.sync_copy(idx_hbm.at[pl.ds(flat0, _IDXSTAGE)], idx_v)
            for u in range(4):
                copies = []
                for k in range(_MAJ):
                    o = u * _MAJ * s1 + k * s1
                    copies.append(pltpu.async_copy(
                        table_hbm.at[idx_v.at[pl.ds(o, 128)]],
                        rows_v.at[k, pl.ds(0, 128)],
                        sem,
                    ))
                    copies.append(pltpu.async_copy(
                        table_hbm.at[idx_v.at[pl.ds(o + 128, 72)]],
                        rows_v.at[k, pl.ds(128, 72)],
                        sem,
                    ))
                for c in copies:
                    c.wait()
                maj = pl.multiple_of(maj_base + t * 16 + u * _MAJ, _MAJ)
                pltpu.sync_copy(rows_v, out_hbm.at[pl.ds(maj, _MAJ)])
            return carry

        lax.fori_loop(0, n_outer, body, 0)

    return gather_kernel(table, idx1d)


def kernel(x, table):
    s0, s1 = x.shape
    idx1d = x.reshape(-1).astype(jnp.int32)
    return _gather(table, idx1d, s0, s1)


# speedup vs baseline: 2.4955x; 1.6329x over previous
import functools

import jax
import jax.numpy as jnp
from jax import lax
from jax.experimental import pallas as pl
from jax.experimental.pallas import tpu as pltpu
from jax.experimental.pallas import tpu_sc as plsc

D_MODEL = 64
_NC = 2
_NS = 16
_NW = _NC * _NS
_SUB = 128
_K = 8
_CHUNK = _K * _SUB


@functools.partial(jax.jit, static_argnames=("n_iter",))
def _gather(table, idx1d, n_iter):
    b = idx1d.shape[0]
    b_per_w = b // _NW
    mesh = plsc.VectorSubcoreMesh(core_axis_name="c", subcore_axis_name="s")

    @functools.partial(
        pl.kernel,
        mesh=mesh,
        compiler_params=pltpu.CompilerParams(use_tc_tiling_on_sc=False),
        out_type=jax.ShapeDtypeStruct((b, 2 * D_MODEL), jnp.float32),
        scratch_types=[
            pltpu.VMEM((_CHUNK,), jnp.int32),
            pltpu.VMEM((_CHUNK, D_MODEL), jnp.float32),
            pltpu.SemaphoreType.DMA,
        ],
    )
    def gather_kernel(table_hbm, idx_hbm, out_hbm, idx_v, rows_v, sem):
        wid = lax.axis_index("s") * _NC + lax.axis_index("c")
        base = wid * b_per_w

        def body(i, carry):
            off = pl.multiple_of(base + i * _CHUNK, _CHUNK)
            pltpu.sync_copy(idx_hbm.at[pl.ds(off, _CHUNK)], idx_v)
            copies = [
                pltpu.async_copy(
                    table_hbm.at[idx_v.at[pl.ds(j * _SUB, _SUB)]],
                    rows_v.at[pl.ds(j * _SUB, _SUB)],
                    sem,
                )
                for j in range(_K)
            ]
            for c in copies:
                c.wait()
            pltpu.sync_copy(
                rows_v,
                out_hbm.at[pl.ds(off, _CHUNK), pl.ds(0, D_MODEL)],
            )
            return carry

        lax.fori_loop(0, n_iter, body, 0)

    return gather_kernel(table, idx1d)


def kernel(x, table):
    s0, s1 = x.shape
    b = s0 * s1
    idx1d = x.reshape(-1).astype(jnp.int32)
    n_iter = b // (_NW * _CHUNK)
    out128 = _gather(table, idx1d, n_iter)
    return out128.reshape(s0, s1, 2 * D_MODEL)[:, :, :D_MODEL]
